# baseline (speedup 1.0000x reference)
import functools

import jax
import jax.numpy as jnp
from jax import lax
from jax.experimental import pallas as pl
from jax.experimental.pallas import tpu as pltpu
from jax.experimental.pallas import tpu_sc as plsc

WIDTH = 32
CH = 128
NC = 2
NS = 16
NW = NC * NS
NB = 8
CW = 16


def _mesh():
    return plsc.VectorSubcoreMesh(core_axis_name="c", subcore_axis_name="s")


def _worker_id():
    return lax.axis_index("s") * NC + lax.axis_index("c")


def _make_gather(n, ep):
    cpw = ep // (CH * NW)

    @functools.partial(
        pl.kernel,
        mesh=_mesh(),
        out_type=jax.ShapeDtypeStruct((ep, WIDTH), jnp.float32),
        compiler_params=pltpu.CompilerParams(use_tc_tiling_on_sc=False),
        scratch_types=[
            pltpu.VMEM((cpw * CH,), jnp.int32),
            pltpu.VMEM((NB, CH, WIDTH), jnp.float32),
            [pltpu.SemaphoreType.DMA] * NB,
            [pltpu.SemaphoreType.DMA] * NB,
        ],
    )
    def gather(h_hbm, src_hbm, out_hbm, idx_v, rows_v, gsem, osem):
        wid = _worker_id()
        base = wid * cpw
        pltpu.sync_copy(src_hbm.at[pl.ds(base * CH, cpw * CH)], idx_v)

        def body(g, carry):
            descs = []
            for b in range(NB):
                c = g * NB + b
                descs.append(pltpu.async_copy(
                    h_hbm.at[idx_v.at[pl.ds(c * CH, CH)]], rows_v.at[b],
                    gsem[b]))
            odescs = []
            for b in range(NB):
                c = g * NB + b
                descs[b].wait()
                odescs.append(pltpu.async_copy(
                    rows_v.at[b], out_hbm.at[pl.ds((base + c) * CH, CH)],
                    osem[b]))
            for b in range(NB):
                odescs[b].wait()
            return carry

        lax.fori_loop(0, cpw // NB, body, 0)

    return gather


def _make_scatter(np_, ep, with_counts):
    cpw = ep // (CH * NW)
    rpt = np_ // NS

    out_type = [jax.ShapeDtypeStruct((NC * np_, WIDTH), jnp.float32)]
    scratch = [
        pltpu.VMEM_SHARED((np_, WIDTH), jnp.float32),
        pltpu.VMEM((cpw * CH,), jnp.int32),
        pltpu.VMEM((NB, CH, WIDTH), jnp.float32),
        pltpu.VMEM((rpt, WIDTH), jnp.float32),
        [pltpu.SemaphoreType.DMA] * NB,
    ]
    if with_counts:
        out_type.append(jax.ShapeDtypeStruct((NC * np_, CW), jnp.float32))
        scratch += [
            pltpu.VMEM_SHARED((np_, CW), jnp.float32),
            pltpu.VMEM((CH, CW), jnp.float32),
            pltpu.VMEM((rpt, CW), jnp.float32),
        ]

    @functools.partial(
        pl.kernel,
        mesh=_mesh(),
        out_type=tuple(out_type),
        compiler_params=pltpu.CompilerParams(use_tc_tiling_on_sc=False),
        scratch_types=scratch,
    )
    def scatter(rows_hbm, dst_hbm, z32_hbm, *rest):
        if with_counts:
            (z16_hbm, ones_hbm, out_hbm, cnt_hbm, acc_sh, idx_v,
             rows_v, stage_v, rsem, cacc_sh, ones_v, cstage_v) = rest
        else:
            (out_hbm, acc_sh, idx_v, rows_v, stage_v, rsem) = rest
        cid = lax.axis_index("c")
        sid = lax.axis_index("s")
        wid = sid * NC + cid
        r0 = sid * rpt
        base = wid * cpw

        pltpu.sync_copy(z32_hbm.at[pl.ds(r0, rpt)], stage_v)
        pltpu.sync_copy(stage_v, acc_sh.at[pl.ds(r0, rpt)])
        pltpu.sync_copy(dst_hbm.at[pl.ds(base * CH, cpw * CH)], idx_v)
        if with_counts:
            pltpu.sync_copy(z16_hbm.at[pl.ds(r0, rpt)], cstage_v)
            pltpu.sync_copy(cstage_v, cacc_sh.at[pl.ds(r0, rpt)])
            pltpu.sync_copy(ones_hbm, ones_v)
        plsc.subcore_barrier()

        def body(g, carry):
            descs = []
            for b in range(NB):
                c = g * NB + b
                descs.append(pltpu.async_copy(
                    rows_hbm.at[pl.ds((base + c) * CH, CH)], rows_v.at[b],
                    rsem[b]))
            for b in range(NB):
                c = g * NB + b
                descs[b].wait()
                pltpu.sync_copy(rows_v.at[b],
                                acc_sh.at[idx_v.at[pl.ds(c * CH, CH)]],
                                add=True)
                if with_counts:
                    pltpu.sync_copy(ones_v,
                                    cacc_sh.at[idx_v.at[pl.ds(c * CH, CH)]],
                                    add=True)
            return carry

        lax.fori_loop(0, cpw // NB, body, 0)
        plsc.subcore_barrier()

        pltpu.sync_copy(acc_sh.at[pl.ds(r0, rpt)], stage_v)
        pltpu.sync_copy(stage_v, out_hbm.at[pl.ds(cid * np_ + r0, rpt)])
        if with_counts:
            pltpu.sync_copy(cacc_sh.at[pl.ds(r0, rpt)], cstage_v)
            pltpu.sync_copy(cstage_v, cnt_hbm.at[pl.ds(cid * np_ + r0, rpt)])

    return scatter


def _lin_body(x_ref, w_ref, b_ref, o_ref, *, relu):
    y = jnp.dot(x_ref[...], w_ref[...], preferred_element_type=jnp.float32)
    y = y + b_ref[...]
    o_ref[...] = jnp.maximum(y, 0.0) if relu else y


def _tc_linear(x, w, b, relu=False):
    n, _ = x.shape
    fo = w.shape[1]
    return pl.pallas_call(
        functools.partial(_lin_body, relu=relu),
        out_shape=jax.ShapeDtypeStruct((n, fo), jnp.float32),
    )(x, w, b.reshape(1, fo))


def _msg_body(ea_ref, hs_ref, k1w, k1b, k2w, k2b, k3w, rep_ref, sel_ref,
              kb_ref, o_ref, *, eb):
    a = jnp.dot(ea_ref[...], k1w[...], preferred_element_type=jnp.float32)
    a = jnp.maximum(a + k1b[...], 0.0)
    a = jnp.dot(a, k2w[...], preferred_element_type=jnp.float32)
    a = jnp.maximum(a + k2b[...], 0.0)
    w = jnp.dot(a, k3w[...], preferred_element_type=jnp.float32)
    pk = 128 // WIDTH
    sub = eb // pk
    hp = hs_ref[...]
    h = jnp.concatenate(
        [hp[:, j * WIDTH:(j + 1) * WIDTH] for j in range(pk)], axis=0)
    hrep = jnp.dot(h, rep_ref[...], preferred_element_type=jnp.float32)
    msg = jnp.dot(hrep * w, sel_ref[...], preferred_element_type=jnp.float32)
    msg = msg + jnp.dot(h, kb_ref[...], preferred_element_type=jnp.float32)
    o_ref[...] = jnp.concatenate(
        [msg[j * sub:(j + 1) * sub, :] for j in range(pk)], axis=1)


def _tc_msg(edge_attr, h_src, k1_w, k1_b, k2_w, k2_b, k3_w, k3_b, eb=640):
    e, ki = edge_attr.shape
    epk = h_src.shape[0]
    kw2 = k2_w.shape[1]
    kw1 = k1_w.shape[1]
    ww = WIDTH * WIDTH
    ebk = eb * WIDTH // 128
    grid = e // eb
    full = lambda i: (0, 0)
    j = jnp.arange(ww)
    rep = (j[None, :] // WIDTH == jnp.arange(WIDTH)[:, None]).astype(jnp.float32)
    sel = (j[:, None] % WIDTH == jnp.arange(WIDTH)[None, :]).astype(jnp.float32)
    return pl.pallas_call(
        functools.partial(_msg_body, eb=eb),
        grid=(grid,),
        in_specs=[
            pl.BlockSpec((eb, ki), lambda i: (i, 0)),
            pl.BlockSpec((ebk, 128), lambda i: (i, 0)),
            pl.BlockSpec(k1_w.shape, full),
            pl.BlockSpec((1, kw1), full),
            pl.BlockSpec(k2_w.shape, full),
            pl.BlockSpec((1, kw2), full),
            pl.BlockSpec(k3_w.shape, full),
            pl.BlockSpec((WIDTH, ww), full),
            pl.BlockSpec((ww, WIDTH), full),
            pl.BlockSpec((WIDTH, WIDTH), full),
        ],
        out_specs=pl.BlockSpec((ebk, 128), lambda i: (i, 0)),
        out_shape=jax.ShapeDtypeStruct((epk, 128), jnp.float32),
    )(edge_attr, h_src, k1_w, k1_b.reshape(1, kw1), k2_w, k2_b.reshape(1, kw2),
      k3_w, rep, sel, k3_b.reshape(WIDTH, WIDTH))


def _update_body(p_ref, c_ref, h_ref, rw_ref, cb_ref, o_ref, *, n, np_, relu):
    cnt = jnp.maximum(c_ref[0:n, 0:1] + c_ref[np_:np_ + n, 0:1], 1.0)
    agg = (p_ref[0:n, :] + p_ref[np_:np_ + n, :]) / cnt
    y = agg + jnp.dot(h_ref[...], rw_ref[...],
                      preferred_element_type=jnp.float32) + cb_ref[...]
    o_ref[...] = jnp.maximum(y, 0.0) if relu else y


def _tc_update(parts, cnts, h, root_w, conv_b, relu):
    n = h.shape[0]
    np_ = parts.shape[0] // NC
    return pl.pallas_call(
        functools.partial(_update_body, n=n, np_=np_, relu=relu),
        out_shape=jax.ShapeDtypeStruct((n, WIDTH), jnp.float32),
    )(parts, cnts, h, root_w, conv_b.reshape(1, WIDTH))


def _head_body(h_ref, w2_ref, b2_ref, w3_ref, b3_ref, o_ref):
    a = jnp.dot(h_ref[...], w2_ref[...], preferred_element_type=jnp.float32)
    a = jnp.maximum(a + b2_ref[...], 0.0)
    o_ref[...] = jnp.dot(a, w3_ref[...],
                         preferred_element_type=jnp.float32) + b3_ref[...]


def _tc_head(h, fc2_w, fc2_b, fc3_w, fc3_b):
    n = h.shape[0]
    kw = fc2_w.shape[1]
    return pl.pallas_call(
        _head_body,
        out_shape=jax.ShapeDtypeStruct((n, 1), jnp.float32),
    )(h, fc2_w, fc2_b.reshape(1, kw), fc3_w, fc3_b.reshape(1, 1))


def kernel(x, edge_index, edge_attr, fc1_w, fc1_b, k1_w, k1_b, k2_w, k2_b,
           k3_w, k3_b, root_w, conv_b, fc2_w, fc2_b, fc3_w, fc3_b):
    n = x.shape[0]
    e = edge_attr.shape[0]
    src = edge_index[0].astype(jnp.int32)
    dst = edge_index[1].astype(jnp.int32)

    grain = CH * NW * NB
    ep = ((e + grain - 1) // grain) * grain
    np_ = ((n + NS) // NS) * NS
    pad = ep - e
    srcp = jnp.concatenate([src, jnp.zeros((pad,), jnp.int32)])
    dstp = jnp.concatenate([dst, jnp.full((pad,), n, jnp.int32)])

    gather = _make_gather(n, ep)
    scatter_c = _make_scatter(np_, ep, with_counts=True)
    scatter = _make_scatter(np_, ep, with_counts=False)

    zeros32 = jnp.zeros((np_, WIDTH), jnp.float32)
    zeros16 = jnp.zeros((np_, CW), jnp.float32)
    ones16 = jnp.ones((CH, CW), jnp.float32)

    h = _tc_linear(x, fc1_w, fc1_b)

    depth = 2
    cnts = None
    eb = 640
    pk = 128 // WIDTH
    ki = edge_attr.shape[1]
    ea_perm = edge_attr.reshape(e // eb, eb // pk, pk, ki)
    ea_perm = ea_perm.transpose(0, 2, 1, 3).reshape(e, ki)

    for k in range(depth):
        h_src = gather(h, srcp)
        hs_packed = h_src.reshape(ep * WIDTH // 128, 128)
        msg_packed = _tc_msg(ea_perm, hs_packed, k1_w, k1_b, k2_w, k2_b,
                             k3_w, k3_b)
        msg = msg_packed.reshape(ep, WIDTH)
        if k == 0:
            parts, cnts = scatter_c(msg, dstp, zeros32, zeros16, ones16)
        else:
            (parts,) = scatter(msg, dstp, zeros32)
        h = _tc_update(parts, cnts, h, root_w, conv_b, relu=(k != depth - 1))

    return _tc_head(h, fc2_w, fc2_b, fc3_w, fc3_b)


# speedup vs baseline: 1.0024x; 1.0024x over previous
import functools

import jax
import jax.numpy as jnp
from jax import lax
from jax.experimental import pallas as pl
from jax.experimental.pallas import tpu as pltpu
from jax.experimental.pallas import tpu_sc as plsc

WIDTH = 32
CH = 128
NC = 2
NS = 16
NW = NC * NS
NB = 8
CW = 16


def _mesh():
    return plsc.VectorSubcoreMesh(core_axis_name="c", subcore_axis_name="s")


def _worker_id():
    return lax.axis_index("s") * NC + lax.axis_index("c")


def _make_gather(n, ep):
    cpw = ep // (CH * NW)

    @functools.partial(
        pl.kernel,
        mesh=_mesh(),
        out_type=jax.ShapeDtypeStruct((ep, WIDTH), jnp.float32),
        compiler_params=pltpu.CompilerParams(use_tc_tiling_on_sc=False),
        scratch_types=[
            pltpu.VMEM((cpw, CH), jnp.int32),
            pltpu.VMEM((NB, CH, WIDTH), jnp.float32),
            [pltpu.SemaphoreType.DMA] * NB,
            [pltpu.SemaphoreType.DMA] * NB,
        ],
    )
    def gather(h_hbm, src_hbm, out_hbm, idx_v, rows_v, gsem, osem):
        wid = _worker_id()
        base = wid * cpw
        pltpu.sync_copy(src_hbm.at[pl.ds(base, cpw)], idx_v)

        def body(g, carry):
            descs = []
            for b in range(NB):
                c = g * NB + b
                descs.append(pltpu.async_copy(
                    h_hbm.at[idx_v.at[c]], rows_v.at[b], gsem[b]))
            odescs = []
            for b in range(NB):
                c = g * NB + b
                descs[b].wait()
                odescs.append(pltpu.async_copy(
                    rows_v.at[b], out_hbm.at[pl.ds((base + c) * CH, CH)],
                    osem[b]))
            for b in range(NB):
                odescs[b].wait()
            return carry

        lax.fori_loop(0, cpw // NB, body, 0)

    return gather


def _make_scatter(np_, ep, with_counts):
    cpw = ep // (CH * NW)
    rpt = np_ // NS

    out_type = [jax.ShapeDtypeStruct((NC * np_, WIDTH), jnp.float32)]
    scratch = [
        pltpu.VMEM_SHARED((np_, WIDTH), jnp.float32),
        pltpu.VMEM((cpw, CH), jnp.int32),
        pltpu.VMEM((NB, CH, WIDTH), jnp.float32),
        pltpu.VMEM((rpt, WIDTH), jnp.float32),
        [pltpu.SemaphoreType.DMA] * NB,
    ]
    if with_counts:
        out_type.append(jax.ShapeDtypeStruct((NC * np_, CW), jnp.float32))
        scratch += [
            pltpu.VMEM_SHARED((np_, CW), jnp.float32),
            pltpu.VMEM((CH, CW), jnp.float32),
            pltpu.VMEM((rpt, CW), jnp.float32),
        ]

    @functools.partial(
        pl.kernel,
        mesh=_mesh(),
        out_type=tuple(out_type),
        compiler_params=pltpu.CompilerParams(use_tc_tiling_on_sc=False),
        scratch_types=scratch,
    )
    def scatter(rows_hbm, dst_hbm, z32_hbm, *rest):
        if with_counts:
            (z16_hbm, ones_hbm, out_hbm, cnt_hbm, acc_sh, idx_v,
             rows_v, stage_v, rsem, cacc_sh, ones_v, cstage_v) = rest
        else:
            (out_hbm, acc_sh, idx_v, rows_v, stage_v, rsem) = rest
        cid = lax.axis_index("c")
        sid = lax.axis_index("s")
        wid = sid * NC + cid
        r0 = sid * rpt
        base = wid * cpw

        pltpu.sync_copy(z32_hbm.at[pl.ds(r0, rpt)], stage_v)
        pltpu.sync_copy(stage_v, acc_sh.at[pl.ds(r0, rpt)])
        pltpu.sync_copy(dst_hbm.at[pl.ds(base, cpw)], idx_v)
        if with_counts:
            pltpu.sync_copy(z16_hbm.at[pl.ds(r0, rpt)], cstage_v)
            pltpu.sync_copy(cstage_v, cacc_sh.at[pl.ds(r0, rpt)])
            pltpu.sync_copy(ones_hbm, ones_v)
        plsc.subcore_barrier()

        def body(g, carry):
            descs = []
            for b in range(NB):
                c = g * NB + b
                descs.append(pltpu.async_copy(
                    rows_hbm.at[pl.ds((base + c) * CH, CH)], rows_v.at[b],
                    rsem[b]))
            for b in range(NB):
                c = g * NB + b
                descs[b].wait()
                pltpu.sync_copy(rows_v.at[b], acc_sh.at[idx_v.at[c]], add=True)
                if with_counts:
                    pltpu.sync_copy(ones_v, cacc_sh.at[idx_v.at[c]], add=True)
            return carry

        lax.fori_loop(0, cpw // NB, body, 0)
        plsc.subcore_barrier()

        pltpu.sync_copy(acc_sh.at[pl.ds(r0, rpt)], stage_v)
        pltpu.sync_copy(stage_v, out_hbm.at[pl.ds(cid * np_ + r0, rpt)])
        if with_counts:
            pltpu.sync_copy(cacc_sh.at[pl.ds(r0, rpt)], cstage_v)
            pltpu.sync_copy(cstage_v, cnt_hbm.at[pl.ds(cid * np_ + r0, rpt)])

    return scatter


def _lin_body(x_ref, w_ref, b_ref, o_ref, *, relu):
    y = jnp.dot(x_ref[...], w_ref[...], preferred_element_type=jnp.float32)
    y = y + b_ref[...]
    o_ref[...] = jnp.maximum(y, 0.0) if relu else y


def _tc_linear(x, w, b, relu=False):
    n, _ = x.shape
    fo = w.shape[1]
    return pl.pallas_call(
        functools.partial(_lin_body, relu=relu),
        out_shape=jax.ShapeDtypeStruct((n, fo), jnp.float32),
    )(x, w, b.reshape(1, fo))


def _msg_body(ea_ref, hs_ref, k1w, k1b, k2w, k2b, k3w, rep_ref, sel_ref,
              kb_ref, o_ref, *, eb):
    a = jnp.dot(ea_ref[...], k1w[...], preferred_element_type=jnp.float32)
    a = jnp.maximum(a + k1b[...], 0.0)
    a = jnp.dot(a, k2w[...], preferred_element_type=jnp.float32)
    a = jnp.maximum(a + k2b[...], 0.0)
    w = jnp.dot(a, k3w[...], preferred_element_type=jnp.float32)
    pk = 128 // WIDTH
    sub = eb // pk
    hp = hs_ref[...]
    h = jnp.concatenate(
        [hp[:, j * WIDTH:(j + 1) * WIDTH] for j in range(pk)], axis=0)
    hrep = jnp.dot(h, rep_ref[...], preferred_element_type=jnp.float32)
    msg = jnp.dot(hrep * w, sel_ref[...], preferred_element_type=jnp.float32)
    msg = msg + jnp.dot(h, kb_ref[...], preferred_element_type=jnp.float32)
    o_ref[...] = jnp.concatenate(
        [msg[j * sub:(j + 1) * sub, :] for j in range(pk)], axis=1)


def _tc_msg(edge_attr, h_src, k1_w, k1_b, k2_w, k2_b, k3_w, k3_b, eb=640):
    e, ki = edge_attr.shape
    epk = h_src.shape[0]
    kw2 = k2_w.shape[1]
    kw1 = k1_w.shape[1]
    ww = WIDTH * WIDTH
    ebk = eb * WIDTH // 128
    grid = e // eb
    full = lambda i: (0, 0)
    j = jnp.arange(ww)
    rep = (j[None, :] // WIDTH == jnp.arange(WIDTH)[:, None]).astype(jnp.float32)
    sel = (j[:, None] % WIDTH == jnp.arange(WIDTH)[None, :]).astype(jnp.float32)
    return pl.pallas_call(
        functools.partial(_msg_body, eb=eb),
        grid=(grid,),
        in_specs=[
            pl.BlockSpec((eb, ki), lambda i: (i, 0)),
            pl.BlockSpec((ebk, 128), lambda i: (i, 0)),
            pl.BlockSpec(k1_w.shape, full),
            pl.BlockSpec((1, kw1), full),
            pl.BlockSpec(k2_w.shape, full),
            pl.BlockSpec((1, kw2), full),
            pl.BlockSpec(k3_w.shape, full),
            pl.BlockSpec((WIDTH, ww), full),
            pl.BlockSpec((ww, WIDTH), full),
            pl.BlockSpec((WIDTH, WIDTH), full),
        ],
        out_specs=pl.BlockSpec((ebk, 128), lambda i: (i, 0)),
        out_shape=jax.ShapeDtypeStruct((epk, 128), jnp.float32),
    )(edge_attr, h_src, k1_w, k1_b.reshape(1, kw1), k2_w, k2_b.reshape(1, kw2),
      k3_w, rep, sel, k3_b.reshape(WIDTH, WIDTH))


def _update_body(p_ref, c_ref, h_ref, rw_ref, cb_ref, o_ref, *, n, np_, relu):
    cnt = jnp.maximum(c_ref[0:n, 0:1] + c_ref[np_:np_ + n, 0:1], 1.0)
    agg = (p_ref[0:n, :] + p_ref[np_:np_ + n, :]) / cnt
    y = agg + jnp.dot(h_ref[...], rw_ref[...],
                      preferred_element_type=jnp.float32) + cb_ref[...]
    o_ref[...] = jnp.maximum(y, 0.0) if relu else y


def _tc_update(parts, cnts, h, root_w, conv_b, relu):
    n = h.shape[0]
    np_ = parts.shape[0] // NC
    return pl.pallas_call(
        functools.partial(_update_body, n=n, np_=np_, relu=relu),
        out_shape=jax.ShapeDtypeStruct((n, WIDTH), jnp.float32),
    )(parts, cnts, h, root_w, conv_b.reshape(1, WIDTH))


def _head_body(h_ref, w2_ref, b2_ref, w3_ref, b3_ref, o_ref):
    a = jnp.dot(h_ref[...], w2_ref[...], preferred_element_type=jnp.float32)
    a = jnp.maximum(a + b2_ref[...], 0.0)
    o_ref[...] = jnp.dot(a, w3_ref[...],
                         preferred_element_type=jnp.float32) + b3_ref[...]


def _tc_head(h, fc2_w, fc2_b, fc3_w, fc3_b):
    n = h.shape[0]
    kw = fc2_w.shape[1]
    return pl.pallas_call(
        _head_body,
        out_shape=jax.ShapeDtypeStruct((n, 1), jnp.float32),
    )(h, fc2_w, fc2_b.reshape(1, kw), fc3_w, fc3_b.reshape(1, 1))


def kernel(x, edge_index, edge_attr, fc1_w, fc1_b, k1_w, k1_b, k2_w, k2_b,
           k3_w, k3_b, root_w, conv_b, fc2_w, fc2_b, fc3_w, fc3_b):
    n = x.shape[0]
    e = edge_attr.shape[0]
    src = edge_index[0].astype(jnp.int32)
    dst = edge_index[1].astype(jnp.int32)

    grain = CH * NW * NB
    ep = ((e + grain - 1) // grain) * grain
    np_ = ((n + NS) // NS) * NS
    pad = ep - e
    srcp = jnp.concatenate([src, jnp.zeros((pad,), jnp.int32)])
    srcp = srcp.reshape(ep // CH, CH)
    dstp = jnp.concatenate([dst, jnp.full((pad,), n, jnp.int32)])
    dstp = dstp.reshape(ep // CH, CH)

    gather = _make_gather(n, ep)
    scatter_c = _make_scatter(np_, ep, with_counts=True)
    scatter = _make_scatter(np_, ep, with_counts=False)

    zeros32 = jnp.zeros((np_, WIDTH), jnp.float32)
    zeros16 = jnp.zeros((np_, CW), jnp.float32)
    ones16 = jnp.ones((CH, CW), jnp.float32)

    h = _tc_linear(x, fc1_w, fc1_b)

    depth = 2
    cnts = None
    eb = 640
    pk = 128 // WIDTH
    ki = edge_attr.shape[1]
    ea_perm = edge_attr.reshape(e // eb, eb // pk, pk, ki)
    ea_perm = ea_perm.transpose(0, 2, 1, 3).reshape(e, ki)

    for k in range(depth):
        h_src = gather(h, srcp)
        hs_packed = h_src.reshape(ep * WIDTH // 128, 128)
        msg_packed = _tc_msg(ea_perm, hs_packed, k1_w, k1_b, k2_w, k2_b,
                             k3_w, k3_b)
        msg = msg_packed.reshape(ep, WIDTH)
        if k == 0:
            parts, cnts = scatter_c(msg, dstp, zeros32, zeros16, ones16)
        else:
            (parts,) = scatter(msg, dstp, zeros32)
        h = _tc_update(parts, cnts, h, root_w, conv_b, relu=(k != depth - 1))

    return _tc_head(h, fc2_w, fc2_b, fc3_w, fc3_b)
